# baseline (device time: 14157 ns/iter reference)
import jax
import jax.numpy as jnp
from jax import lax
from jax.experimental import pallas as pl
from jax.experimental.pallas import tpu as pltpu

N_DEV = 4


def kernel(table, idx):
    v_per, d = table.shape
    n = idx.shape[0]
    half = n // 2

    def body(table_ref, idx_ref, out_ref, send_buf, recv_buf,
             send_sems, recv_sems):
        my_pos = lax.axis_index("i")
        p0 = my_pos ^ 1
        p1 = 3 - my_pos

        barrier_sem = pltpu.get_barrier_semaphore()
        for nbr in [p0, p1]:
            pl.semaphore_signal(
                barrier_sem, inc=1,
                device_id=(nbr,), device_id_type=pl.DeviceIdType.MESH,
            )

        tbl = table_ref[:, :].astype(jnp.bfloat16)
        lidx = idx_ref[:, :] - my_pos * v_per

        def partial(lo, hi):
            iota = lax.broadcasted_iota(jnp.int32, (v_per, hi - lo), 0)
            oh = (iota == lidx[:, lo:hi]).astype(jnp.bfloat16)
            p = lax.dot_general(
                oh, tbl, (((0,), (0,)), ((), ())),
                preferred_element_type=jnp.float32,
            )
            return p.astype(jnp.bfloat16)

        def exchange(slot, partner):
            return pltpu.make_async_remote_copy(
                src_ref=send_buf.at[slot],
                dst_ref=recv_buf.at[slot],
                send_sem=send_sems.at[slot],
                recv_sem=recv_sems.at[slot],
                device_id=(partner,),
                device_id_type=pl.DeviceIdType.MESH,
            )

        acc_a = partial(0, half)
        send_buf[0, :, :] = acc_a
        pl.semaphore_wait(barrier_sem, 2)
        r0a = exchange(0, p0)
        r0a.start()

        acc_b = partial(half, n)
        send_buf[1, :, :] = acc_b
        r0b = exchange(1, p0)
        r0b.start()

        r0a.wait_recv()
        acc_a = acc_a + recv_buf[0, :, :]
        send_buf[2, :, :] = acc_a
        r1a = exchange(2, p1)
        r1a.start()

        r0b.wait_recv()
        acc_b = acc_b + recv_buf[1, :, :]
        send_buf[3, :, :] = acc_b
        r1b = exchange(3, p1)
        r1b.start()

        r1a.wait_recv()
        out_ref[pl.ds(0, half), :] = acc_a + recv_buf[2, :, :]
        r1b.wait_recv()
        out_ref[pl.ds(half, half), :] = acc_b + recv_buf[3, :, :]

        r0a.wait_send()
        r0b.wait_send()
        r1a.wait_send()
        r1b.wait_send()

    return pl.pallas_call(
        body,
        out_shape=jax.ShapeDtypeStruct((n, d), jnp.bfloat16),
        in_specs=[
            pl.BlockSpec(memory_space=pltpu.VMEM),
            pl.BlockSpec(memory_space=pltpu.VMEM),
        ],
        out_specs=pl.BlockSpec(memory_space=pltpu.VMEM),
        scratch_shapes=[
            pltpu.VMEM((4, half, d), jnp.bfloat16),
            pltpu.VMEM((4, half, d), jnp.bfloat16),
            pltpu.SemaphoreType.DMA((4,)),
            pltpu.SemaphoreType.DMA((4,)),
        ],
        compiler_params=pltpu.CompilerParams(collective_id=0),
    )(table, idx.reshape(1, n))


# device time: 13300 ns/iter; 1.0644x vs baseline; 1.0644x over previous
import jax
import jax.numpy as jnp
from jax import lax
from jax.experimental import pallas as pl
from jax.experimental.pallas import tpu as pltpu

N_DEV = 4
K = 4


def kernel(table, idx):
    v_per, d = table.shape
    n = idx.shape[0]
    c = n // K

    def body(table_hbm, idx_ref, out_ref, tbl_vmem, send_buf, recv_buf,
             send_sems, recv_sems, load_sem):
        my_pos = lax.axis_index("i")
        p0 = my_pos ^ 1
        p1 = 3 - my_pos

        barrier_sem = pltpu.get_barrier_semaphore()
        for nbr in [p0, p1]:
            pl.semaphore_signal(
                barrier_sem, inc=1,
                device_id=(nbr,), device_id_type=pl.DeviceIdType.MESH,
            )

        load = pltpu.make_async_copy(table_hbm, tbl_vmem, load_sem)
        load.start()

        lidx = idx_ref[:, :] - my_pos * v_per
        load.wait()
        tbl = tbl_vmem[:, :].astype(jnp.bfloat16)

        def partial(lo, hi):
            iota = lax.broadcasted_iota(jnp.int32, (v_per, hi - lo), 0)
            oh = (iota == lidx[:, lo:hi]).astype(jnp.bfloat16)
            p = lax.dot_general(
                oh, tbl, (((0,), (0,)), ((), ())),
                preferred_element_type=jnp.float32,
            )
            return p.astype(jnp.bfloat16)

        def exchange(slot, partner):
            return pltpu.make_async_remote_copy(
                src_ref=send_buf.at[slot],
                dst_ref=recv_buf.at[slot],
                send_sem=send_sems.at[slot],
                recv_sem=recv_sems.at[slot],
                device_id=(partner,),
                device_id_type=pl.DeviceIdType.MESH,
            )

        accs = []
        r0 = []
        for k in range(K):
            acc_k = partial(k * c, (k + 1) * c)
            send_buf[k, :, :] = acc_k
            if k == 0:
                pl.semaphore_wait(barrier_sem, 2)
            rk = exchange(k, p0)
            rk.start()
            accs.append(acc_k)
            r0.append(rk)

        r1 = []
        for k in range(K):
            r0[k].wait_recv()
            accs[k] = accs[k] + recv_buf[k, :, :]
            send_buf[K + k, :, :] = accs[k]
            rk = exchange(K + k, p1)
            rk.start()
            r1.append(rk)

        for k in range(K):
            r1[k].wait_recv()
            out_ref[pl.ds(k * c, c), :] = accs[k] + recv_buf[K + k, :, :]

        for rk in r0 + r1:
            rk.wait_send()

    return pl.pallas_call(
        body,
        out_shape=jax.ShapeDtypeStruct((n, d), jnp.bfloat16),
        in_specs=[
            pl.BlockSpec(memory_space=pl.ANY),
            pl.BlockSpec(memory_space=pltpu.VMEM),
        ],
        out_specs=pl.BlockSpec(memory_space=pltpu.VMEM),
        scratch_shapes=[
            pltpu.VMEM((v_per, d), jnp.float32),
            pltpu.VMEM((2 * K, c, d), jnp.bfloat16),
            pltpu.VMEM((2 * K, c, d), jnp.bfloat16),
            pltpu.SemaphoreType.DMA((2 * K,)),
            pltpu.SemaphoreType.DMA((2 * K,)),
            pltpu.SemaphoreType.DMA,
        ],
        compiler_params=pltpu.CompilerParams(collective_id=0),
    )(table, idx.reshape(1, n))


# device time: 12659 ns/iter; 1.1183x vs baseline; 1.0506x over previous
import jax
import jax.numpy as jnp
from jax import lax
from jax.experimental import pallas as pl
from jax.experimental.pallas import tpu as pltpu

N_DEV = 4
K = 4


def kernel(table, idx):
    v_per, d = table.shape
    n = idx.shape[0]
    c = n // K

    def body(table_hbm, idx_hbm, out_hbm, tbl_vmem, idx_vmem, store_buf,
             send_buf, recv_buf, send_sems, recv_sems, store_sems,
             load_sem, idx_sem):
        my_pos = lax.axis_index("i")
        p0 = my_pos ^ 1
        p1 = 3 - my_pos

        barrier_sem = pltpu.get_barrier_semaphore()
        for nbr in [p0, p1]:
            pl.semaphore_signal(
                barrier_sem, inc=1,
                device_id=(nbr,), device_id_type=pl.DeviceIdType.MESH,
            )

        load = pltpu.make_async_copy(table_hbm, tbl_vmem, load_sem)
        load.start()
        iload = pltpu.make_async_copy(idx_hbm, idx_vmem, idx_sem)
        iload.start()

        iload.wait()
        lidx = idx_vmem[:, :] - my_pos * v_per
        load.wait()
        tbl = tbl_vmem[:, :].astype(jnp.bfloat16)

        def partial(lo, hi):
            iota = lax.broadcasted_iota(jnp.int32, (v_per, hi - lo), 0)
            oh = (iota == lidx[:, lo:hi]).astype(jnp.bfloat16)
            p = lax.dot_general(
                oh, tbl, (((0,), (0,)), ((), ())),
                preferred_element_type=jnp.float32,
            )
            return p.astype(jnp.bfloat16)

        def exchange(slot, partner):
            return pltpu.make_async_remote_copy(
                src_ref=send_buf.at[slot],
                dst_ref=recv_buf.at[slot],
                send_sem=send_sems.at[slot],
                recv_sem=recv_sems.at[slot],
                device_id=(partner,),
                device_id_type=pl.DeviceIdType.MESH,
            )

        accs = []
        r0 = []
        for k in range(K):
            acc_k = partial(k * c, (k + 1) * c)
            send_buf[k, :, :] = acc_k
            if k == 0:
                pl.semaphore_wait(barrier_sem, 2)
            rk = exchange(k, p0)
            rk.start()
            accs.append(acc_k)
            r0.append(rk)

        r1 = []
        for k in range(K):
            r0[k].wait_recv()
            accs[k] = accs[k] + recv_buf[k, :, :]
            send_buf[K + k, :, :] = accs[k]
            rk = exchange(K + k, p1)
            rk.start()
            r1.append(rk)

        stores = []
        for k in range(K):
            r1[k].wait_recv()
            store_buf[k, :, :] = accs[k] + recv_buf[K + k, :, :]
            st = pltpu.make_async_copy(
                store_buf.at[k],
                out_hbm.at[pl.ds(k * c, c), :],
                store_sems.at[k],
            )
            st.start()
            stores.append(st)

        for st in stores:
            st.wait()
        for rk in r0 + r1:
            rk.wait_send()

    call = pl.pallas_call(
        body,
        out_shape=jax.ShapeDtypeStruct((n, d), jnp.bfloat16),
        in_specs=[
            pl.BlockSpec(memory_space=pl.ANY),
            pl.BlockSpec(memory_space=pl.ANY),
        ],
        out_specs=pl.BlockSpec(memory_space=pl.ANY),
        scratch_shapes=[
            pltpu.VMEM((v_per, d), jnp.float32),
            pltpu.VMEM((1, n), jnp.int32),
            pltpu.VMEM((K, c, d), jnp.bfloat16),
            pltpu.VMEM((2 * K, c, d), jnp.bfloat16),
            pltpu.VMEM((2 * K, c, d), jnp.bfloat16),
            pltpu.SemaphoreType.DMA((2 * K,)),
            pltpu.SemaphoreType.DMA((2 * K,)),
            pltpu.SemaphoreType.DMA((K,)),
            pltpu.SemaphoreType.DMA,
            pltpu.SemaphoreType.DMA,
        ],
        compiler_params=pltpu.CompilerParams(collective_id=0),
    )
    table = pltpu.with_memory_space_constraint(table, pltpu.MemorySpace.HBM)
    idx2 = pltpu.with_memory_space_constraint(
        idx.reshape(1, n), pltpu.MemorySpace.HBM
    )
    return call(table, idx2)
